# Initial kernel scaffold; baseline (speedup 1.0000x reference)
import jax
import jax.numpy as jnp
from jax.experimental import pallas as pl


def kernel(x, W0, att_src0, att_dst0, b0, W1, att_src1, att_dst1, b1, W2, att_src2, att_dst2, b2, Wc, bc):
    raise NotImplementedError("write your pallas kernel here")


# speedup vs baseline: 23.7500x; 23.7500x over previous
import functools

import jax
import jax.numpy as jnp
from jax.experimental import pallas as pl
from jax.experimental.pallas import tpu as pltpu

_KNN = 16
_BM = 256
_NEG = -1e30


def _norm_body(x_ref, o_ref):
    xi = x_ref[0]
    c = xi - jnp.mean(xi, axis=0, keepdims=True)
    o_ref[0] = c * (0.999999 / jnp.max(jnp.abs(c)))


def _knn_body(xr_ref, xf_ref, m_ref):
    xr = xr_ref[0]
    xf = xf_ref[0]
    bm, _ = xr.shape
    n = xf.shape[0]
    j = pl.program_id(1)
    sqr = jnp.sum(xr * xr, axis=1, keepdims=True)
    ones = jnp.ones((1, xf.shape[1]), dtype=jnp.float32)
    sqf = jax.lax.dot_general(
        ones, xf * xf, (((1,), (1,)), ((), ())),
        preferred_element_type=jnp.float32)
    cross = jax.lax.dot_general(
        xr, xf, (((1,), (1,)), ((), ())),
        preferred_element_type=jnp.float32)
    d2 = sqr - 2.0 * cross + sqf
    colid = jax.lax.broadcasted_iota(jnp.int32, (bm, n), 1)
    rowid = jax.lax.broadcasted_iota(jnp.int32, (bm, n), 0) + j * bm
    d2 = jnp.where(colid == rowid, 1e18, d2)

    def body(_, carry):
        d2c, sel = carry
        mn = jnp.min(d2c, axis=1, keepdims=True)
        cand = jnp.where(d2c == mn, colid, n)
        pick = colid == jnp.min(cand, axis=1, keepdims=True)
        return jnp.where(pick, 3e38, d2c), jnp.where(pick, 1.0, sel)

    _, sel = jax.lax.fori_loop(
        0, _KNN, body, (d2, jnp.zeros((bm, n), jnp.float32)))
    m_ref[0] = sel


def _mm_body(x_ref, w_ref, o_ref):
    o_ref[...] = jnp.dot(x_ref[...], w_ref[...],
                         preferred_element_type=jnp.float32)


def _gat_body(hr_ref, hf_ref, m_ref, as_ref, ad_ref, b_ref, o_ref, *, relu):
    hr = hr_ref[0]
    hf = hf_ref[0]
    bm = hr.shape[0]
    n = hf.shape[0]
    j = pl.program_id(1)
    es = jax.lax.dot_general(
        as_ref[...], hf, (((1,), (1,)), ((), ())),
        preferred_element_type=jnp.float32)
    ed = jax.lax.dot_general(
        hr, ad_ref[...], (((1,), (1,)), ((), ())),
        preferred_element_type=jnp.float32)
    logits = es + ed
    logits = jnp.where(logits >= 0.0, logits, 0.2 * logits)
    colid = jax.lax.broadcasted_iota(jnp.int32, (bm, n), 1)
    rowid = jax.lax.broadcasted_iota(jnp.int32, (bm, n), 0) + j * bm
    valid = (m_ref[0] > 0.0) | (colid == rowid)
    logits = jnp.where(valid, logits, _NEG)
    mx = jnp.max(logits, axis=1, keepdims=True)
    ex = jnp.where(valid, jnp.exp(logits - mx), 0.0)
    alpha = ex / jnp.sum(ex, axis=1, keepdims=True)
    out = jnp.dot(alpha, hf, preferred_element_type=jnp.float32) + b_ref[...]
    if relu:
        out = jnp.maximum(out, 0.0)
    o_ref[0] = out


def _head_body(h_ref, wc_ref, bc_ref, o_ref):
    p = jnp.max(h_ref[0], axis=0, keepdims=True)
    o_ref[0] = jnp.dot(p, wc_ref[...],
                       preferred_element_type=jnp.float32) + bc_ref[...]


def kernel(x, W0, att_src0, att_dst0, b0, W1, att_src1, att_dst1, b1,
           W2, att_src2, att_dst2, b2, Wc, bc):
    B, N, D = x.shape
    f32 = jnp.float32

    xn = pl.pallas_call(
        _norm_body,
        grid=(B,),
        in_specs=[pl.BlockSpec((1, N, D), lambda i: (i, 0, 0))],
        out_specs=pl.BlockSpec((1, N, D), lambda i: (i, 0, 0)),
        out_shape=jax.ShapeDtypeStruct((B, N, D), f32),
    )(x)

    mask = pl.pallas_call(
        _knn_body,
        grid=(B, N // _BM),
        in_specs=[
            pl.BlockSpec((1, _BM, D), lambda i, j: (i, j, 0)),
            pl.BlockSpec((1, N, D), lambda i, j: (i, 0, 0)),
        ],
        out_specs=pl.BlockSpec((1, _BM, N), lambda i, j: (i, j, 0)),
        out_shape=jax.ShapeDtypeStruct((B, N, N), f32),
    )(x, x)

    def feat_mm(h, W):
        h2 = h.reshape(B * N, D)
        out = pl.pallas_call(
            _mm_body,
            grid=(B * N // 512,),
            in_specs=[
                pl.BlockSpec((512, D), lambda i: (i, 0)),
                pl.BlockSpec((D, D), lambda i: (0, 0)),
            ],
            out_specs=pl.BlockSpec((512, D), lambda i: (i, 0)),
            out_shape=jax.ShapeDtypeStruct((B * N, D), f32),
        )(h2, W)
        return out.reshape(B, N, D)

    def gat_layer(h, a_s, a_d, b, relu):
        return pl.pallas_call(
            functools.partial(_gat_body, relu=relu),
            grid=(B, N // _BM),
            in_specs=[
                pl.BlockSpec((1, _BM, D), lambda i, j: (i, j, 0)),
                pl.BlockSpec((1, N, D), lambda i, j: (i, 0, 0)),
                pl.BlockSpec((1, _BM, N), lambda i, j: (i, j, 0)),
                pl.BlockSpec((1, D), lambda i, j: (0, 0)),
                pl.BlockSpec((1, D), lambda i, j: (0, 0)),
                pl.BlockSpec((1, D), lambda i, j: (0, 0)),
            ],
            out_specs=pl.BlockSpec((1, _BM, D), lambda i, j: (i, j, 0)),
            out_shape=jax.ShapeDtypeStruct((B, N, D), f32),
        )(h, h, mask, a_s.reshape(1, D), a_d.reshape(1, D), b.reshape(1, D))

    h = xn
    for (W, a_s, a_d, b, relu) in (
            (W0, att_src0, att_dst0, b0, True),
            (W1, att_src1, att_dst1, b1, True),
            (W2, att_src2, att_dst2, b2, False)):
        hw = feat_mm(h, W)
        h = gat_layer(hw, a_s, a_d, b, relu)

    C = Wc.shape[1]
    wc_pad = jnp.zeros((D, 128), f32).at[:, :C].set(Wc)
    bc_pad = jnp.zeros((1, 128), f32).at[0, :C].set(bc)
    out = pl.pallas_call(
        _head_body,
        grid=(B,),
        in_specs=[
            pl.BlockSpec((1, N, D), lambda i: (i, 0, 0)),
            pl.BlockSpec((D, 128), lambda i: (0, 0)),
            pl.BlockSpec((1, 128), lambda i: (0, 0)),
        ],
        out_specs=pl.BlockSpec((1, 1, 128), lambda i: (i, 0, 0)),
        out_shape=jax.ShapeDtypeStruct((B, 1, 128), f32),
    )(h, wc_pad, bc_pad)
    return out[:, 0, :C]


# speedup vs baseline: 23.9036x; 1.0065x over previous
import functools

import jax
import jax.numpy as jnp
from jax.experimental import pallas as pl
from jax.experimental.pallas import tpu as pltpu

_KNN = 16
_BM = 256
_NEG = -1e30


def _norm_body(x_ref, o_ref):
    xi = x_ref[0]
    c = xi - jnp.mean(xi, axis=0, keepdims=True)
    o_ref[0] = c * (0.999999 / jnp.max(jnp.abs(c)))


def _knn_body(xr_ref, xf_ref, m_ref):
    xr = xr_ref[0]
    xf = xf_ref[0]
    bm, _ = xr.shape
    n = xf.shape[0]
    j = pl.program_id(1)
    sqr = jnp.sum(xr * xr, axis=1, keepdims=True)
    sqf = jnp.sum(xf * xf, axis=1, keepdims=True).reshape(1, n)
    cross = jax.lax.dot_general(
        xr, xf, (((1,), (1,)), ((), ())),
        preferred_element_type=jnp.float32)
    d2 = (sqr + sqf) - 2.0 * cross
    colid = jax.lax.broadcasted_iota(jnp.int32, (bm, n), 1)
    rowid = jax.lax.broadcasted_iota(jnp.int32, (bm, n), 0) + j * bm
    d2 = jnp.where(colid == rowid, 1e18, d2)

    def body(_, carry):
        d2c, sel = carry
        mn = jnp.min(d2c, axis=1, keepdims=True)
        cand = jnp.where(d2c == mn, colid, n)
        pick = colid == jnp.min(cand, axis=1, keepdims=True)
        return jnp.where(pick, 3e38, d2c), jnp.where(pick, 1.0, sel)

    _, sel = jax.lax.fori_loop(
        0, _KNN, body, (d2, jnp.zeros((bm, n), jnp.float32)))
    m_ref[0] = sel


def _mm_body(x_ref, w_ref, o_ref):
    o_ref[...] = jnp.dot(x_ref[...], w_ref[...],
                         preferred_element_type=jnp.float32)


def _gat_body(hr_ref, hf_ref, m_ref, as_ref, ad_ref, b_ref, o_ref, *, relu):
    hr = hr_ref[0]
    hf = hf_ref[0]
    bm = hr.shape[0]
    n = hf.shape[0]
    j = pl.program_id(1)
    es = jax.lax.dot_general(
        as_ref[...], hf, (((1,), (1,)), ((), ())),
        preferred_element_type=jnp.float32)
    ed = jax.lax.dot_general(
        hr, ad_ref[...], (((1,), (1,)), ((), ())),
        preferred_element_type=jnp.float32)
    logits = es + ed
    logits = jnp.where(logits >= 0.0, logits, 0.2 * logits)
    colid = jax.lax.broadcasted_iota(jnp.int32, (bm, n), 1)
    rowid = jax.lax.broadcasted_iota(jnp.int32, (bm, n), 0) + j * bm
    valid = (m_ref[0] > 0.0) | (colid == rowid)
    logits = jnp.where(valid, logits, _NEG)
    mx = jnp.max(logits, axis=1, keepdims=True)
    ex = jnp.where(valid, jnp.exp(logits - mx), 0.0)
    alpha = ex / jnp.sum(ex, axis=1, keepdims=True)
    out = jnp.dot(alpha, hf, preferred_element_type=jnp.float32) + b_ref[...]
    if relu:
        out = jnp.maximum(out, 0.0)
    o_ref[0] = out


def _head_body(h_ref, wc_ref, bc_ref, o_ref):
    p = jnp.max(h_ref[0], axis=0, keepdims=True)
    o_ref[0] = jnp.dot(p, wc_ref[...],
                       preferred_element_type=jnp.float32) + bc_ref[...]


def kernel(x, W0, att_src0, att_dst0, b0, W1, att_src1, att_dst1, b1,
           W2, att_src2, att_dst2, b2, Wc, bc):
    B, N, D = x.shape
    f32 = jnp.float32

    xn = pl.pallas_call(
        _norm_body,
        grid=(B,),
        in_specs=[pl.BlockSpec((1, N, D), lambda i: (i, 0, 0))],
        out_specs=pl.BlockSpec((1, N, D), lambda i: (i, 0, 0)),
        out_shape=jax.ShapeDtypeStruct((B, N, D), f32),
    )(x)

    mask = pl.pallas_call(
        _knn_body,
        grid=(B, N // _BM),
        in_specs=[
            pl.BlockSpec((1, _BM, D), lambda i, j: (i, j, 0)),
            pl.BlockSpec((1, N, D), lambda i, j: (i, 0, 0)),
        ],
        out_specs=pl.BlockSpec((1, _BM, N), lambda i, j: (i, j, 0)),
        out_shape=jax.ShapeDtypeStruct((B, N, N), f32),
    )(x, x)

    def feat_mm(h, W):
        h2 = h.reshape(B * N, D)
        out = pl.pallas_call(
            _mm_body,
            grid=(B * N // 512,),
            in_specs=[
                pl.BlockSpec((512, D), lambda i: (i, 0)),
                pl.BlockSpec((D, D), lambda i: (0, 0)),
            ],
            out_specs=pl.BlockSpec((512, D), lambda i: (i, 0)),
            out_shape=jax.ShapeDtypeStruct((B * N, D), f32),
        )(h2, W)
        return out.reshape(B, N, D)

    def gat_layer(h, a_s, a_d, b, relu):
        return pl.pallas_call(
            functools.partial(_gat_body, relu=relu),
            grid=(B, N // _BM),
            in_specs=[
                pl.BlockSpec((1, _BM, D), lambda i, j: (i, j, 0)),
                pl.BlockSpec((1, N, D), lambda i, j: (i, 0, 0)),
                pl.BlockSpec((1, _BM, N), lambda i, j: (i, j, 0)),
                pl.BlockSpec((1, D), lambda i, j: (0, 0)),
                pl.BlockSpec((1, D), lambda i, j: (0, 0)),
                pl.BlockSpec((1, D), lambda i, j: (0, 0)),
            ],
            out_specs=pl.BlockSpec((1, _BM, D), lambda i, j: (i, j, 0)),
            out_shape=jax.ShapeDtypeStruct((B, N, D), f32),
        )(h, h, mask, a_s.reshape(1, D), a_d.reshape(1, D), b.reshape(1, D))

    h = xn
    for (W, a_s, a_d, b, relu) in (
            (W0, att_src0, att_dst0, b0, True),
            (W1, att_src1, att_dst1, b1, True),
            (W2, att_src2, att_dst2, b2, False)):
        hw = feat_mm(h, W)
        h = gat_layer(hw, a_s, a_d, b, relu)

    C = Wc.shape[1]
    wc_pad = jnp.zeros((D, 128), f32).at[:, :C].set(Wc)
    bc_pad = jnp.zeros((1, 128), f32).at[0, :C].set(bc)
    out = pl.pallas_call(
        _head_body,
        grid=(B,),
        in_specs=[
            pl.BlockSpec((1, N, D), lambda i: (i, 0, 0)),
            pl.BlockSpec((D, 128), lambda i: (0, 0)),
            pl.BlockSpec((1, 128), lambda i: (0, 0)),
        ],
        out_specs=pl.BlockSpec((1, 1, 128), lambda i: (i, 0, 0)),
        out_shape=jax.ShapeDtypeStruct((B, 1, 128), f32),
    )(h, wc_pad, bc_pad)
    return out[:, 0, :C]


# speedup vs baseline: 24.9295x; 1.0429x over previous
import functools

import jax
import jax.numpy as jnp
from jax.experimental import pallas as pl
from jax.experimental.pallas import tpu as pltpu

_KNN = 16
_BM = 256
_NEG = -1e30


def _norm_mm_body(x_ref, w_ref, o_ref):
    xi = x_ref[0]
    c = xi - jnp.mean(xi, axis=0, keepdims=True)
    xn = c * (0.999999 / jnp.max(jnp.abs(c)))
    o_ref[0] = jnp.dot(xn, w_ref[...], preferred_element_type=jnp.float32)


def _knn_body(xr_ref, xf_ref, m_ref):
    xr = xr_ref[0]
    xf = xf_ref[0]
    bm, _ = xr.shape
    n = xf.shape[0]
    j = pl.program_id(1)
    sqr = jnp.sum(xr * xr, axis=1, keepdims=True)
    sqf = jnp.sum(xf * xf, axis=1, keepdims=True).reshape(1, n)
    cross = jax.lax.dot_general(
        xr, xf, (((1,), (1,)), ((), ())),
        preferred_element_type=jnp.float32)
    d2 = (sqr + sqf) - 2.0 * cross
    colid = jax.lax.broadcasted_iota(jnp.int32, (bm, n), 1)
    rowid = jax.lax.broadcasted_iota(jnp.int32, (bm, n), 0) + j * bm
    d2 = jnp.where(colid == rowid, 1e18, d2)

    def body(_, carry):
        d2c, sel = carry
        mn = jnp.min(d2c, axis=1, keepdims=True)
        cand = jnp.where(d2c == mn, colid, n)
        pick = colid == jnp.min(cand, axis=1, keepdims=True)
        return jnp.where(pick, 3e38, d2c), jnp.where(pick, 1.0, sel)

    _, sel = jax.lax.fori_loop(
        0, _KNN, body, (d2, jnp.zeros((bm, n), jnp.float32)))
    m_ref[0] = sel.astype(jnp.bfloat16)


def _gat_body(hr_ref, hf_ref, m_ref, as_ref, ad_ref, b_ref, *rest, relu,
              fuse_next):
    if fuse_next:
        wn_ref, o_ref = rest
    else:
        (o_ref,) = rest
    hr = hr_ref[0]
    hf = hf_ref[0]
    bm = hr.shape[0]
    n = hf.shape[0]
    j = pl.program_id(1)
    es = jax.lax.dot_general(
        as_ref[...], hf, (((1,), (1,)), ((), ())),
        preferred_element_type=jnp.float32)
    ed = jax.lax.dot_general(
        hr, ad_ref[...], (((1,), (1,)), ((), ())),
        preferred_element_type=jnp.float32)
    logits = es + ed
    logits = jnp.where(logits >= 0.0, logits, 0.2 * logits)
    colid = jax.lax.broadcasted_iota(jnp.int32, (bm, n), 1)
    rowid = jax.lax.broadcasted_iota(jnp.int32, (bm, n), 0) + j * bm
    valid = (m_ref[0] > 0) | (colid == rowid)
    logits = jnp.where(valid, logits, _NEG)
    mx = jnp.max(logits, axis=1, keepdims=True)
    ex = jnp.where(valid, jnp.exp(logits - mx), 0.0)
    alpha = ex / jnp.sum(ex, axis=1, keepdims=True)
    out = jnp.dot(alpha, hf, preferred_element_type=jnp.float32) + b_ref[...]
    if relu:
        out = jnp.maximum(out, 0.0)
    if fuse_next:
        out = jnp.dot(out, wn_ref[...], preferred_element_type=jnp.float32)
    o_ref[0] = out


def _head_body(h_ref, wc_ref, bc_ref, o_ref):
    p = jnp.max(h_ref[0], axis=0, keepdims=True)
    o_ref[0] = jnp.dot(p, wc_ref[...],
                       preferred_element_type=jnp.float32) + bc_ref[...]


def kernel(x, W0, att_src0, att_dst0, b0, W1, att_src1, att_dst1, b1,
           W2, att_src2, att_dst2, b2, Wc, bc):
    B, N, D = x.shape
    f32 = jnp.float32
    params = pltpu.CompilerParams(
        dimension_semantics=("parallel", "parallel"))

    h = pl.pallas_call(
        _norm_mm_body,
        grid=(B,),
        in_specs=[
            pl.BlockSpec((1, N, D), lambda i: (i, 0, 0)),
            pl.BlockSpec((D, D), lambda i: (0, 0)),
        ],
        out_specs=pl.BlockSpec((1, N, D), lambda i: (i, 0, 0)),
        out_shape=jax.ShapeDtypeStruct((B, N, D), f32),
        compiler_params=pltpu.CompilerParams(
            dimension_semantics=("parallel",)),
    )(x, W0)

    mask = pl.pallas_call(
        _knn_body,
        grid=(B, N // _BM),
        in_specs=[
            pl.BlockSpec((1, _BM, D), lambda i, j: (i, j, 0)),
            pl.BlockSpec((1, N, D), lambda i, j: (i, 0, 0)),
        ],
        out_specs=pl.BlockSpec((1, _BM, N), lambda i, j: (i, j, 0)),
        out_shape=jax.ShapeDtypeStruct((B, N, N), jnp.bfloat16),
        compiler_params=params,
    )(x, x)

    def gat_layer(h, a_s, a_d, b, relu, W_next):
        fuse = W_next is not None
        in_specs = [
            pl.BlockSpec((1, _BM, D), lambda i, j: (i, j, 0)),
            pl.BlockSpec((1, N, D), lambda i, j: (i, 0, 0)),
            pl.BlockSpec((1, _BM, N), lambda i, j: (i, j, 0)),
            pl.BlockSpec((1, D), lambda i, j: (0, 0)),
            pl.BlockSpec((1, D), lambda i, j: (0, 0)),
            pl.BlockSpec((1, D), lambda i, j: (0, 0)),
        ]
        args = [h, h, mask, a_s.reshape(1, D), a_d.reshape(1, D),
                b.reshape(1, D)]
        if fuse:
            in_specs.append(pl.BlockSpec((D, D), lambda i, j: (0, 0)))
            args.append(W_next)
        return pl.pallas_call(
            functools.partial(_gat_body, relu=relu, fuse_next=fuse),
            grid=(B, N // _BM),
            in_specs=in_specs,
            out_specs=pl.BlockSpec((1, _BM, D), lambda i, j: (i, j, 0)),
            out_shape=jax.ShapeDtypeStruct((B, N, D), f32),
            compiler_params=params,
        )(*args)

    h = gat_layer(h, att_src0, att_dst0, b0, True, W1)
    h = gat_layer(h, att_src1, att_dst1, b1, True, W2)
    h = gat_layer(h, att_src2, att_dst2, b2, False, None)

    C = Wc.shape[1]
    wc_pad = jnp.zeros((D, 128), f32).at[:, :C].set(Wc)
    bc_pad = jnp.zeros((1, 128), f32).at[0, :C].set(bc)
    out = pl.pallas_call(
        _head_body,
        grid=(B,),
        in_specs=[
            pl.BlockSpec((1, N, D), lambda i: (i, 0, 0)),
            pl.BlockSpec((D, 128), lambda i: (0, 0)),
            pl.BlockSpec((1, 128), lambda i: (0, 0)),
        ],
        out_specs=pl.BlockSpec((1, 1, 128), lambda i: (i, 0, 0)),
        out_shape=jax.ShapeDtypeStruct((B, 1, 128), f32),
        compiler_params=pltpu.CompilerParams(
            dimension_semantics=("parallel",)),
    )(h, wc_pad, bc_pad)
    return out[:, 0, :C]


# speedup vs baseline: 26.6092x; 1.0674x over previous
import functools

import jax
import jax.numpy as jnp
from jax.experimental import pallas as pl
from jax.experimental.pallas import tpu as pltpu

_KNN = 16
_BM = 512
_NEG = -1e30


def _norm_mm_body(x_ref, w_ref, o_ref):
    xi = x_ref[0]
    c = xi - jnp.mean(xi, axis=0, keepdims=True)
    xn = c * (0.999999 / jnp.max(jnp.abs(c)))
    o_ref[0] = jnp.dot(xn, w_ref[...], preferred_element_type=jnp.float32)


def _knn_body(xf_ref, m_ref):
    xf = xf_ref[0]
    n = xf.shape[0]
    bm = _BM
    j = pl.program_id(1)
    xr = xf_ref[0, pl.ds(j * bm, bm), :]
    sqr = jnp.sum(xr * xr, axis=1, keepdims=True)
    sqf = jnp.sum(xf * xf, axis=1, keepdims=True).reshape(1, n)
    cross = jax.lax.dot_general(
        xr, xf, (((1,), (1,)), ((), ())),
        preferred_element_type=jnp.float32)
    d2 = (sqr + sqf) - 2.0 * cross
    colid = jax.lax.broadcasted_iota(jnp.int32, (bm, n), 1)
    rowid = jax.lax.broadcasted_iota(jnp.int32, (bm, n), 0) + j * bm
    d2 = jnp.where(colid == rowid, 1e18, d2)

    def body(_, carry):
        d2c, sel = carry
        mn = jnp.min(d2c, axis=1, keepdims=True)
        cand = jnp.where(d2c == mn, colid, n)
        pick = colid == jnp.min(cand, axis=1, keepdims=True)
        return jnp.where(pick, 3e38, d2c), jnp.where(pick, 1.0, sel)

    _, sel = jax.lax.fori_loop(
        0, _KNN, body, (d2, jnp.zeros((bm, n), jnp.float32)))
    m_ref[0] = sel.astype(jnp.bfloat16)


def _gat_body(hf_ref, m_ref, as_ref, ad_ref, b_ref, *rest, relu,
              fuse_next):
    if fuse_next:
        wn_ref, o_ref = rest
    else:
        (o_ref,) = rest
    hf = hf_ref[0]
    bm = _BM
    n = hf.shape[0]
    j = pl.program_id(1)
    hr = hf_ref[0, pl.ds(j * bm, bm), :]
    es = jax.lax.dot_general(
        as_ref[...], hf, (((1,), (1,)), ((), ())),
        preferred_element_type=jnp.float32)
    ed = jax.lax.dot_general(
        hr, ad_ref[...], (((1,), (1,)), ((), ())),
        preferred_element_type=jnp.float32)
    logits = es + ed
    logits = jnp.where(logits >= 0.0, logits, 0.2 * logits)
    colid = jax.lax.broadcasted_iota(jnp.int32, (bm, n), 1)
    rowid = jax.lax.broadcasted_iota(jnp.int32, (bm, n), 0) + j * bm
    valid = (m_ref[0] > 0) | (colid == rowid)
    logits = jnp.where(valid, logits, _NEG)
    mx = jnp.max(logits, axis=1, keepdims=True)
    ex = jnp.where(valid, jnp.exp(logits - mx), 0.0)
    alpha = ex / jnp.sum(ex, axis=1, keepdims=True)
    out = jnp.dot(alpha, hf, preferred_element_type=jnp.float32) + b_ref[...]
    if relu:
        out = jnp.maximum(out, 0.0)
    if fuse_next:
        out = jnp.dot(out, wn_ref[...], preferred_element_type=jnp.float32)
    o_ref[0] = out


def _head_body(h_ref, wc_ref, bc_ref, o_ref):
    p = jnp.max(h_ref[0], axis=0, keepdims=True)
    o_ref[0] = jnp.dot(p, wc_ref[...],
                       preferred_element_type=jnp.float32) + bc_ref[...]


def kernel(x, W0, att_src0, att_dst0, b0, W1, att_src1, att_dst1, b1,
           W2, att_src2, att_dst2, b2, Wc, bc):
    B, N, D = x.shape
    f32 = jnp.float32
    params = pltpu.CompilerParams(
        dimension_semantics=("parallel", "parallel"))

    h = pl.pallas_call(
        _norm_mm_body,
        grid=(B,),
        in_specs=[
            pl.BlockSpec((1, N, D), lambda i: (i, 0, 0)),
            pl.BlockSpec((D, D), lambda i: (0, 0)),
        ],
        out_specs=pl.BlockSpec((1, N, D), lambda i: (i, 0, 0)),
        out_shape=jax.ShapeDtypeStruct((B, N, D), f32),
        compiler_params=pltpu.CompilerParams(
            dimension_semantics=("parallel",)),
    )(x, W0)

    mask = pl.pallas_call(
        _knn_body,
        grid=(B, N // _BM),
        in_specs=[
            pl.BlockSpec((1, N, D), lambda i, j: (i, 0, 0)),
        ],
        out_specs=pl.BlockSpec((1, _BM, N), lambda i, j: (i, j, 0)),
        out_shape=jax.ShapeDtypeStruct((B, N, N), jnp.bfloat16),
        compiler_params=params,
    )(x)

    def gat_layer(h, a_s, a_d, b, relu, W_next):
        fuse = W_next is not None
        in_specs = [
            pl.BlockSpec((1, N, D), lambda i, j: (i, 0, 0)),
            pl.BlockSpec((1, _BM, N), lambda i, j: (i, j, 0)),
            pl.BlockSpec((1, D), lambda i, j: (0, 0)),
            pl.BlockSpec((1, D), lambda i, j: (0, 0)),
            pl.BlockSpec((1, D), lambda i, j: (0, 0)),
        ]
        args = [h, mask, a_s.reshape(1, D), a_d.reshape(1, D),
                b.reshape(1, D)]
        if fuse:
            in_specs.append(pl.BlockSpec((D, D), lambda i, j: (0, 0)))
            args.append(W_next)
        return pl.pallas_call(
            functools.partial(_gat_body, relu=relu, fuse_next=fuse),
            grid=(B, N // _BM),
            in_specs=in_specs,
            out_specs=pl.BlockSpec((1, _BM, D), lambda i, j: (i, j, 0)),
            out_shape=jax.ShapeDtypeStruct((B, N, D), f32),
            compiler_params=params,
        )(*args)

    h = gat_layer(h, att_src0, att_dst0, b0, True, W1)
    h = gat_layer(h, att_src1, att_dst1, b1, True, W2)
    h = gat_layer(h, att_src2, att_dst2, b2, False, None)

    C = Wc.shape[1]
    wc_pad = jnp.zeros((D, 128), f32).at[:, :C].set(Wc)
    bc_pad = jnp.zeros((1, 128), f32).at[0, :C].set(bc)
    out = pl.pallas_call(
        _head_body,
        grid=(B,),
        in_specs=[
            pl.BlockSpec((1, N, D), lambda i: (i, 0, 0)),
            pl.BlockSpec((D, 128), lambda i: (0, 0)),
            pl.BlockSpec((1, 128), lambda i: (0, 0)),
        ],
        out_specs=pl.BlockSpec((1, 1, 128), lambda i: (i, 0, 0)),
        out_shape=jax.ShapeDtypeStruct((B, 1, 128), f32),
        compiler_params=pltpu.CompilerParams(
            dimension_semantics=("parallel",)),
    )(h, wc_pad, bc_pad)
    return out[:, 0, :C]


# speedup vs baseline: 36.3103x; 1.3646x over previous
import functools

import jax
import jax.numpy as jnp
from jax.experimental import pallas as pl
from jax.experimental.pallas import tpu as pltpu

_KNN = 16
_BM = 512
_NEG = -1e30


def _norm_mm_body(x_ref, w_ref, o_ref):
    xi = x_ref[0]
    c = xi - jnp.mean(xi, axis=0, keepdims=True)
    xn = c * (0.999999 / jnp.max(jnp.abs(c)))
    o_ref[0] = jnp.dot(xn, w_ref[...], preferred_element_type=jnp.float32)


def _knn_body(xf_ref, m_ref):
    xf = xf_ref[0]
    n = xf.shape[0]
    bm = _BM
    j = pl.program_id(1)
    xr = xf_ref[0, pl.ds(j * bm, bm), :]
    sqr = jnp.sum(xr * xr, axis=1, keepdims=True)
    sqf = jnp.sum(xf * xf, axis=1, keepdims=True).reshape(1, n)
    cross = jax.lax.dot_general(
        xr, xf, (((1,), (1,)), ((), ())),
        preferred_element_type=jnp.float32)
    d2 = (sqr + sqf) - 2.0 * cross
    colid = jax.lax.broadcasted_iota(jnp.int32, (bm, n), 1)
    rowid = jax.lax.broadcasted_iota(jnp.int32, (bm, n), 0) + j * bm
    d2 = jnp.where(colid == rowid, 1e18, d2)

    def body(_, d2c):
        mn = jnp.min(d2c, axis=1, keepdims=True)
        cand = jnp.where(d2c == mn, colid, n)
        pick = colid == jnp.min(cand, axis=1, keepdims=True)
        return jnp.where(pick, 3e38, d2c)

    d2 = jax.lax.fori_loop(0, _KNN, body, d2)
    sel = (d2 == 3e38) | (colid == rowid)
    m_ref[0] = sel.astype(jnp.bfloat16)


def _gat_body(hf_ref, m_ref, as_ref, ad_ref, b_ref, *rest, relu,
              fuse_next):
    if fuse_next:
        wn_ref, o_ref = rest
    else:
        (o_ref,) = rest
    hf = hf_ref[0]
    bm = _BM
    n = hf.shape[0]
    j = pl.program_id(1)
    hr = hf_ref[0, pl.ds(j * bm, bm), :]
    es = jax.lax.dot_general(
        as_ref[...], hf, (((1,), (1,)), ((), ())),
        preferred_element_type=jnp.float32)
    ed = jax.lax.dot_general(
        hr, ad_ref[...], (((1,), (1,)), ((), ())),
        preferred_element_type=jnp.float32)
    logits = es + ed
    logits = jnp.where(logits >= 0.0, logits, 0.2 * logits)
    valid = m_ref[0] > 0
    logits = jnp.where(valid, logits, _NEG)
    mx = jnp.max(logits, axis=1, keepdims=True)
    ex = jnp.exp(logits - mx)
    alpha = ex / jnp.sum(ex, axis=1, keepdims=True)
    out = jnp.dot(alpha, hf, preferred_element_type=jnp.float32) + b_ref[...]
    if relu:
        out = jnp.maximum(out, 0.0)
    if fuse_next:
        out = jnp.dot(out, wn_ref[...], preferred_element_type=jnp.float32)
    o_ref[0] = out


def _head_body(h_ref, wc_ref, bc_ref, o_ref):
    p = jnp.max(h_ref[0], axis=0, keepdims=True)
    o_ref[0] = jnp.dot(p, wc_ref[...],
                       preferred_element_type=jnp.float32) + bc_ref[...]


def kernel(x, W0, att_src0, att_dst0, b0, W1, att_src1, att_dst1, b1,
           W2, att_src2, att_dst2, b2, Wc, bc):
    B, N, D = x.shape
    f32 = jnp.float32
    params = pltpu.CompilerParams(
        dimension_semantics=("parallel", "parallel"))

    h = pl.pallas_call(
        _norm_mm_body,
        grid=(B,),
        in_specs=[
            pl.BlockSpec((1, N, D), lambda i: (i, 0, 0)),
            pl.BlockSpec((D, D), lambda i: (0, 0)),
        ],
        out_specs=pl.BlockSpec((1, N, D), lambda i: (i, 0, 0)),
        out_shape=jax.ShapeDtypeStruct((B, N, D), f32),
        compiler_params=pltpu.CompilerParams(
            dimension_semantics=("parallel",)),
    )(x, W0)

    mask = pl.pallas_call(
        _knn_body,
        grid=(B, N // _BM),
        in_specs=[
            pl.BlockSpec((1, N, D), lambda i, j: (i, 0, 0)),
        ],
        out_specs=pl.BlockSpec((1, _BM, N), lambda i, j: (i, j, 0)),
        out_shape=jax.ShapeDtypeStruct((B, N, N), jnp.bfloat16),
        compiler_params=params,
    )(x)

    def gat_layer(h, a_s, a_d, b, relu, W_next):
        fuse = W_next is not None
        in_specs = [
            pl.BlockSpec((1, N, D), lambda i, j: (i, 0, 0)),
            pl.BlockSpec((1, _BM, N), lambda i, j: (i, j, 0)),
            pl.BlockSpec((1, D), lambda i, j: (0, 0)),
            pl.BlockSpec((1, D), lambda i, j: (0, 0)),
            pl.BlockSpec((1, D), lambda i, j: (0, 0)),
        ]
        args = [h, mask, a_s.reshape(1, D), a_d.reshape(1, D),
                b.reshape(1, D)]
        if fuse:
            in_specs.append(pl.BlockSpec((D, D), lambda i, j: (0, 0)))
            args.append(W_next)
        return pl.pallas_call(
            functools.partial(_gat_body, relu=relu, fuse_next=fuse),
            grid=(B, N // _BM),
            in_specs=in_specs,
            out_specs=pl.BlockSpec((1, _BM, D), lambda i, j: (i, j, 0)),
            out_shape=jax.ShapeDtypeStruct((B, N, D), f32),
            compiler_params=params,
        )(*args)

    h = gat_layer(h, att_src0, att_dst0, b0, True, W1)
    h = gat_layer(h, att_src1, att_dst1, b1, True, W2)
    h = gat_layer(h, att_src2, att_dst2, b2, False, None)

    C = Wc.shape[1]
    wc_pad = jnp.zeros((D, 128), f32).at[:, :C].set(Wc)
    bc_pad = jnp.zeros((1, 128), f32).at[0, :C].set(bc)
    out = pl.pallas_call(
        _head_body,
        grid=(B,),
        in_specs=[
            pl.BlockSpec((1, N, D), lambda i: (i, 0, 0)),
            pl.BlockSpec((D, 128), lambda i: (0, 0)),
            pl.BlockSpec((1, 128), lambda i: (0, 0)),
        ],
        out_specs=pl.BlockSpec((1, 1, 128), lambda i: (i, 0, 0)),
        out_shape=jax.ShapeDtypeStruct((B, 1, 128), f32),
        compiler_params=pltpu.CompilerParams(
            dimension_semantics=("parallel",)),
    )(h, wc_pad, bc_pad)
    return out[:, 0, :C]


# speedup vs baseline: 37.2928x; 1.0271x over previous
import functools

import jax
import jax.numpy as jnp
from jax.experimental import pallas as pl
from jax.experimental.pallas import tpu as pltpu

_KNN = 16
_BM = 1024
_NEG = -1e30


def _norm_mm_body(x_ref, w_ref, o_ref):
    xi = x_ref[0]
    c = xi - jnp.mean(xi, axis=0, keepdims=True)
    xn = c * (0.999999 / jnp.max(jnp.abs(c)))
    o_ref[0] = jnp.dot(xn, w_ref[...], preferred_element_type=jnp.float32)


def _knn_body(xf_ref, m_ref):
    xf = xf_ref[0]
    n = xf.shape[0]
    bm = _BM
    j = pl.program_id(1)
    xr = xf_ref[0, pl.ds(j * bm, bm), :]
    sqr = jnp.sum(xr * xr, axis=1, keepdims=True)
    sqf = jnp.sum(xf * xf, axis=1, keepdims=True).reshape(1, n)
    cross = jax.lax.dot_general(
        xr, xf, (((1,), (1,)), ((), ())),
        preferred_element_type=jnp.float32)
    d2 = (sqr + sqf) - 2.0 * cross
    colid = jax.lax.broadcasted_iota(jnp.int32, (bm, n), 1)
    rowid = jax.lax.broadcasted_iota(jnp.int32, (bm, n), 0) + j * bm
    d2 = jnp.where(colid == rowid, 1e18, d2)

    def body(_, d2c):
        mn = jnp.min(d2c, axis=1, keepdims=True)
        cand = jnp.where(d2c == mn, colid, n)
        pick = colid == jnp.min(cand, axis=1, keepdims=True)
        return jnp.where(pick, 3e38, d2c)

    d2 = jax.lax.fori_loop(0, _KNN, body, d2)
    sel = (d2 == 3e38) | (colid == rowid)
    m_ref[0] = sel.astype(jnp.bfloat16)


def _gat_body(hf_ref, m_ref, as_ref, ad_ref, b_ref, *rest, relu,
              fuse_next):
    if fuse_next:
        wn_ref, o_ref = rest
    else:
        (o_ref,) = rest
    hf = hf_ref[0]
    bm = _BM
    n = hf.shape[0]
    j = pl.program_id(1)
    hr = hf_ref[0, pl.ds(j * bm, bm), :]
    es = jax.lax.dot_general(
        as_ref[...], hf, (((1,), (1,)), ((), ())),
        preferred_element_type=jnp.float32)
    ed = jax.lax.dot_general(
        hr, ad_ref[...], (((1,), (1,)), ((), ())),
        preferred_element_type=jnp.float32)
    logits = es + ed
    logits = jnp.where(logits >= 0.0, logits, 0.2 * logits)
    valid = m_ref[0] > 0
    logits = jnp.where(valid, logits, _NEG)
    mx = jnp.max(logits, axis=1, keepdims=True)
    ex = jnp.exp(logits - mx)
    alpha = ex / jnp.sum(ex, axis=1, keepdims=True)
    out = jnp.dot(alpha, hf, preferred_element_type=jnp.float32) + b_ref[...]
    if relu:
        out = jnp.maximum(out, 0.0)
    if fuse_next:
        out = jnp.dot(out, wn_ref[...], preferred_element_type=jnp.float32)
    o_ref[0] = out


def _head_body(h_ref, wc_ref, bc_ref, o_ref):
    p = jnp.max(h_ref[0], axis=0, keepdims=True)
    o_ref[0] = jnp.dot(p, wc_ref[...],
                       preferred_element_type=jnp.float32) + bc_ref[...]


def kernel(x, W0, att_src0, att_dst0, b0, W1, att_src1, att_dst1, b1,
           W2, att_src2, att_dst2, b2, Wc, bc):
    B, N, D = x.shape
    f32 = jnp.float32
    params = pltpu.CompilerParams(
        dimension_semantics=("parallel", "parallel"))

    h = pl.pallas_call(
        _norm_mm_body,
        grid=(B,),
        in_specs=[
            pl.BlockSpec((1, N, D), lambda i: (i, 0, 0)),
            pl.BlockSpec((D, D), lambda i: (0, 0)),
        ],
        out_specs=pl.BlockSpec((1, N, D), lambda i: (i, 0, 0)),
        out_shape=jax.ShapeDtypeStruct((B, N, D), f32),
        compiler_params=pltpu.CompilerParams(
            dimension_semantics=("parallel",)),
    )(x, W0)

    mask = pl.pallas_call(
        _knn_body,
        grid=(B, N // _BM),
        in_specs=[
            pl.BlockSpec((1, N, D), lambda i, j: (i, 0, 0)),
        ],
        out_specs=pl.BlockSpec((1, _BM, N), lambda i, j: (i, j, 0)),
        out_shape=jax.ShapeDtypeStruct((B, N, N), jnp.bfloat16),
        compiler_params=params,
    )(x)

    def gat_layer(h, a_s, a_d, b, relu, W_next):
        fuse = W_next is not None
        in_specs = [
            pl.BlockSpec((1, N, D), lambda i, j: (i, 0, 0)),
            pl.BlockSpec((1, _BM, N), lambda i, j: (i, j, 0)),
            pl.BlockSpec((1, D), lambda i, j: (0, 0)),
            pl.BlockSpec((1, D), lambda i, j: (0, 0)),
            pl.BlockSpec((1, D), lambda i, j: (0, 0)),
        ]
        args = [h, mask, a_s.reshape(1, D), a_d.reshape(1, D),
                b.reshape(1, D)]
        if fuse:
            in_specs.append(pl.BlockSpec((D, D), lambda i, j: (0, 0)))
            args.append(W_next)
        return pl.pallas_call(
            functools.partial(_gat_body, relu=relu, fuse_next=fuse),
            grid=(B, N // _BM),
            in_specs=in_specs,
            out_specs=pl.BlockSpec((1, _BM, D), lambda i, j: (i, j, 0)),
            out_shape=jax.ShapeDtypeStruct((B, N, D), f32),
            compiler_params=params,
        )(*args)

    h = gat_layer(h, att_src0, att_dst0, b0, True, W1)
    h = gat_layer(h, att_src1, att_dst1, b1, True, W2)
    h = gat_layer(h, att_src2, att_dst2, b2, False, None)

    C = Wc.shape[1]
    wc_pad = jnp.zeros((D, 128), f32).at[:, :C].set(Wc)
    bc_pad = jnp.zeros((1, 128), f32).at[0, :C].set(bc)
    out = pl.pallas_call(
        _head_body,
        grid=(B,),
        in_specs=[
            pl.BlockSpec((1, N, D), lambda i: (i, 0, 0)),
            pl.BlockSpec((D, 128), lambda i: (0, 0)),
            pl.BlockSpec((1, 128), lambda i: (0, 0)),
        ],
        out_specs=pl.BlockSpec((1, 1, 128), lambda i: (i, 0, 0)),
        out_shape=jax.ShapeDtypeStruct((B, 1, 128), f32),
        compiler_params=pltpu.CompilerParams(
            dimension_semantics=("parallel",)),
    )(h, wc_pad, bc_pad)
    return out[:, 0, :C]
